# baseline (speedup 1.0000x reference)
import functools

import jax
import jax.numpy as jnp
from jax import lax
from jax.experimental import pallas as pl
from jax.experimental.pallas import tpu as pltpu
from jax.experimental.pallas import tpu_sc as plsc

NN = 10000
NE = 320000
HID = 64
H2 = 128
OUT_DIM = 3

NC = 2
NS = 16
NW = NC * NS
EPW = NE // NW

GCH = 40
GGC = 2
GGROUP = GCH * GGC
GGN = EPW // GGROUP

SCH = 80
SGN = EPW // SCH

NPAD = 10240
NPS = NPAD // NS

F32 = jnp.float32


def _gather_body(u_hbm, w_hbm, src_hbm, dst_hbm, g_hbm, si, di,
                 a0, b0, a1, b1, g0, g1, s0, s1):
    cid = lax.axis_index("c")
    sid = lax.axis_index("s")
    wid = cid * NS + sid
    pltpu.sync_copy(src_hbm.at[pl.ds(wid * EPW, EPW)], si)
    pltpu.sync_copy(dst_hbm.at[pl.ds(wid * EPW, EPW)], di)
    ebase = wid * EPW

    def issue(g, abuf, bbuf, sem):
        cps = []
        for k in range(GGC):
            off = (g * GGC + k) * GCH
            cps.append(pltpu.async_copy(
                u_hbm.at[si.at[pl.ds(off, GCH)]],
                abuf.at[pl.ds(k * GCH, GCH)], sem))
            cps.append(pltpu.async_copy(
                w_hbm.at[di.at[pl.ds(off, GCH)]],
                bbuf.at[pl.ds(k * GCH, GCH)], sem))
        return cps

    def finish(cps, g, abuf, bbuf, gbuf):
        for cp in cps:
            cp.wait()

        def addrow(r, c2):
            for c in range(HID // 16):
                gbuf[r, pl.ds(c * 16, 16)] = (
                    abuf[r, pl.ds(c * 16, 16)]
                    + bbuf[r, pl.ds(c * 16, 16)])
            return c2

        lax.fori_loop(0, GGROUP, addrow, 0)
        pltpu.sync_copy(gbuf, g_hbm.at[pl.ds(ebase + g * GGROUP, GGROUP)])

    def pair(p, carry):
        cps0 = issue(2 * p, a0, b0, s0)
        cps1 = issue(2 * p + 1, a1, b1, s1)
        finish(cps0, 2 * p, a0, b0, g0)
        finish(cps1, 2 * p + 1, a1, b1, g1)
        return carry

    lax.fori_loop(0, GGN // 2, pair, 0)
    finish(issue(GGN - 1, a0, b0, s0), GGN - 1, a0, b0, g0)


def _sc_gather(u, w, srcp, dstp):
    mesh = plsc.VectorSubcoreMesh(core_axis_name="c", subcore_axis_name="s")
    f = pl.kernel(
        _gather_body,
        out_type=jax.ShapeDtypeStruct((NE, HID), F32),
        mesh=mesh,
        scratch_types=[
            pltpu.VMEM((EPW,), jnp.int32),
            pltpu.VMEM((EPW,), jnp.int32),
            pltpu.VMEM((GGROUP, HID), F32),
            pltpu.VMEM((GGROUP, HID), F32),
            pltpu.VMEM((GGROUP, HID), F32),
            pltpu.VMEM((GGROUP, HID), F32),
            pltpu.VMEM((GGROUP, HID), F32),
            pltpu.VMEM((GGROUP, HID), F32),
            pltpu.SemaphoreType.DMA,
            pltpu.SemaphoreType.DMA,
        ],
        compiler_params=pltpu.CompilerParams(use_tc_tiling_on_sc=False),
    )
    return f(u, w, srcp, dstp)


def _scatter_body(e_hbm, dst_hbm, z_hbm, a0_hbm, a1_hbm,
                  di, eb0, eb1, spmem, lsem0, lsem1, ssem):
    cid = lax.axis_index("c")
    sid = lax.axis_index("s")
    wid = cid * NS + sid
    pltpu.sync_copy(dst_hbm.at[wid], di)
    stripe = pl.ds(sid * NPS, NPS)
    pltpu.sync_copy(z_hbm.at[stripe], spmem.at[stripe])
    plsc.subcore_barrier()

    def erows(j):
        return e_hbm.at[pl.ds(wid * EPW + j * SCH, SCH), pl.ds(0, HID)]

    pltpu.sync_copy(erows(0), eb0)

    def step(p, carry):
        j = 2 * p
        ld1 = pltpu.async_copy(erows(j + 1), eb1, lsem1)
        pltpu.async_copy(eb0, spmem.at[di.at[j]], ssem, add=True).wait()
        ld1.wait()
        ld0 = pltpu.async_copy(erows(j + 2), eb0, lsem0)
        pltpu.async_copy(eb1, spmem.at[di.at[j + 1]], ssem, add=True).wait()
        ld0.wait()
        return carry

    lax.fori_loop(0, (SGN - 1) // 2, step, 0)
    pltpu.async_copy(eb0, spmem.at[di.at[SGN - 1]], ssem, add=True).wait()
    plsc.subcore_barrier()

    @pl.when(cid == 0)
    def _():
        pltpu.sync_copy(spmem.at[stripe], a0_hbm.at[stripe])

    @pl.when(cid == 1)
    def _():
        pltpu.sync_copy(spmem.at[stripe], a1_hbm.at[stripe])


def _sc_scatter(e, dst3, zeros):
    mesh = plsc.VectorSubcoreMesh(core_axis_name="c", subcore_axis_name="s")
    f = pl.kernel(
        _scatter_body,
        out_type=(jax.ShapeDtypeStruct((NPAD, HID), F32),
                  jax.ShapeDtypeStruct((NPAD, HID), F32)),
        mesh=mesh,
        scratch_types=[
            pltpu.VMEM((SGN, SCH), jnp.int32),
            pltpu.VMEM((SCH, HID), F32),
            pltpu.VMEM((SCH, HID), F32),
            pltpu.VMEM_SHARED((NPAD, HID), F32),
            pltpu.SemaphoreType.DMA,
            pltpu.SemaphoreType.DMA,
            pltpu.SemaphoreType.DMA,
        ],
        compiler_params=pltpu.CompilerParams(use_tc_tiling_on_sc=False),
    )
    return f(e, dst3, zeros)


def _dot(a, b):
    return jnp.dot(a, b, preferred_element_type=F32)


def _ln(h, gm, bt):
    m = jnp.mean(h, axis=-1, keepdims=True)
    d = h - m
    v = jnp.mean(d * d, axis=-1, keepdims=True)
    return d * lax.rsqrt(v + 1e-5) * gm + bt


def _pad128(x):
    return jnp.concatenate([x, jnp.zeros_like(x)], axis=-1)


def _edge_enc_body(x_ref, w0, b0, w1, b1, w2, b2, o_ref):
    h = jnp.maximum(_dot(x_ref[...], w0[...]) + b0[...], 0.0)
    h = jnp.maximum(_dot(h, w1[...]) + b1[...], 0.0)
    o_ref[...] = _pad128(_dot(h, w2[...]) + b2[...])


def _node_enc_body(x_ref, w0, b0, w1, b1, w2, b2, puw, pubw,
                   v_ref, uw_ref):
    h = jnp.maximum(_dot(x_ref[...], w0[...]) + b0[...], 0.0)
    h = jnp.maximum(_dot(h, w1[...]) + b1[...], 0.0)
    v = _dot(h, w2[...]) + b2[...]
    v_ref[...] = v
    uw_ref[...] = _dot(v, puw[...]) + pubw[...]


def _edge_block_body(g_ref, e_ref, w0e, w1, b1, w2, b2, gm, bt, o_ref):
    e = e_ref[...][:, :HID]
    h = jnp.maximum(g_ref[...] + _dot(e, w0e[...]), 0.0)
    h = jnp.maximum(_dot(h, w1[...]) + b1[...], 0.0)
    h = _dot(h, w2[...]) + b2[...]
    o_ref[...] = _pad128(e + _ln(h, gm[...], bt[...]))


def _node_block_body(v_ref, a0_ref, a1_ref, w0v, w0a, b0, w1, b1, w2, b2,
                     gm, bt, vo_ref):
    v = v_ref[...]
    a = a0_ref[...] + a1_ref[...]
    h = jnp.maximum(_dot(v, w0v[...]) + _dot(a, w0a[...]) + b0[...], 0.0)
    h = jnp.maximum(_dot(h, w1[...]) + b1[...], 0.0)
    h = _dot(h, w2[...]) + b2[...]
    vo_ref[...] = v + _ln(h, gm[...], bt[...])


def _node_block_proj_body(v_ref, a0_ref, a1_ref, w0v, w0a, b0, w1, b1,
                          w2, b2, gm, bt, puw, pubw,
                          vo_ref, uw_ref):
    v = v_ref[...]
    a = a0_ref[...] + a1_ref[...]
    h = jnp.maximum(_dot(v, w0v[...]) + _dot(a, w0a[...]) + b0[...], 0.0)
    h = jnp.maximum(_dot(h, w1[...]) + b1[...], 0.0)
    h = _dot(h, w2[...]) + b2[...]
    vn = v + _ln(h, gm[...], bt[...])
    vo_ref[...] = vn
    uw_ref[...] = _dot(vn, puw[...]) + pubw[...]


def _dec_body(v_ref, w0, b0, w1, b1, w2, b2, o_ref):
    h = jnp.maximum(_dot(v_ref[...], w0[...]) + b0[...], 0.0)
    h = jnp.maximum(_dot(h, w1[...]) + b1[...], 0.0)
    o_ref[...] = _dot(h, w2[...]) + b2[...]


def _full(shape):
    return pl.BlockSpec(shape, lambda i: (0,) * len(shape))


def _rows(t, d):
    return pl.BlockSpec((t, d), lambda i: (i, 0))


def _tc_call(body, grid, in_specs, out_specs, out_shape):
    return pl.pallas_call(
        body,
        grid=grid,
        in_specs=in_specs,
        out_specs=out_specs,
        out_shape=out_shape,
        compiler_params=pltpu.CompilerParams(
            dimension_semantics=("arbitrary",)),
    )


TE = 4000
TN = 2000


def _mlp_specs(in_dim, hid, out_dim):
    return [_full((in_dim, hid)), _full((1, hid)),
            _full((hid, hid)), _full((1, hid)),
            _full((hid, out_dim)), _full((1, out_dim))]


def _r1(x):
    return x.reshape(1, -1)


def _mlp_args(p):
    return (p["W0"], _r1(p["b0"]), p["W1"], _r1(p["b1"]), p["W2"], _r1(p["b2"]))


def kernel(node_feat, edge_feat, edge_index, params):
    srcp = edge_index[0]
    dstp = edge_index[1]
    dst3 = dstp.reshape(NW, SGN, SCH)

    ebs = params["edge_blocks"]
    nbs = params["node_blocks"]

    esplit = []
    puws = []
    for eb in ebs:
        W0 = eb["mlp"]["W0"]
        esplit.append(W0[2 * HID:])
        puw = jnp.concatenate([W0[:HID], W0[HID:2 * HID]], axis=1)
        pubw = jnp.concatenate(
            [eb["mlp"]["b0"], jnp.zeros((HID,), F32)]).reshape(1, H2)
        puws.append((puw, pubw))
    nsplit = []
    for nb in nbs:
        W0 = nb["mlp"]["W0"]
        nsplit.append((W0[:HID], W0[HID:]))

    ee = params["edge_enc"]
    e = _tc_call(
        _edge_enc_body, (NE // TE,),
        [_rows(TE, 16)] + _mlp_specs(16, HID, HID),
        _rows(TE, H2), jax.ShapeDtypeStruct((NE, H2), F32),
    )(edge_feat, *_mlp_args(ee))

    ne = params["node_enc"]
    v, uw = _tc_call(
        _node_enc_body, (NN // TN,),
        [_rows(TN, 128)] + _mlp_specs(128, HID, HID)
        + [_full((HID, H2)), _full((1, H2))],
        (_rows(TN, HID), _rows(TN, H2)),
        (jax.ShapeDtypeStruct((NN, HID), F32),
         jax.ShapeDtypeStruct((NN, H2), F32)),
    )(node_feat, *_mlp_args(ne), *puws[0])

    zeros = jnp.zeros((NPAD, HID), F32)

    for L in range(len(ebs)):
        eb, nb = ebs[L], nbs[L]
        g = _sc_gather(uw[:, :HID], uw[:, HID:], srcp, dstp)
        e = _tc_call(
            _edge_block_body, (NE // TE,),
            [_rows(TE, HID), _rows(TE, H2)]
            + [_full((HID, HID)), _full((HID, HID)), _full((1, HID)),
               _full((HID, HID)), _full((1, HID)),
               _full((1, HID)), _full((1, HID))],
            _rows(TE, H2), jax.ShapeDtypeStruct((NE, H2), F32),
        )(g, e, esplit[L], eb["mlp"]["W1"], _r1(eb["mlp"]["b1"]),
          eb["mlp"]["W2"], _r1(eb["mlp"]["b2"]), _r1(eb["g"]), _r1(eb["b"]))

        a0, a1 = _sc_scatter(e, dst3, zeros)

        nw = (nsplit[L][0], nsplit[L][1], _r1(nb["mlp"]["b0"]),
              nb["mlp"]["W1"], _r1(nb["mlp"]["b1"]),
              nb["mlp"]["W2"], _r1(nb["mlp"]["b2"]),
              _r1(nb["g"]), _r1(nb["b"]))
        nw_specs = [_full((HID, HID)), _full((HID, HID)), _full((1, HID)),
                    _full((HID, HID)), _full((1, HID)),
                    _full((HID, HID)), _full((1, HID)),
                    _full((1, HID)), _full((1, HID))]
        if L + 1 < len(ebs):
            v, uw = _tc_call(
                _node_block_proj_body, (NN // TN,),
                [_rows(TN, HID), _rows(TN, HID), _rows(TN, HID)] + nw_specs
                + [_full((HID, H2)), _full((1, H2))],
                (_rows(TN, HID), _rows(TN, H2)),
                (jax.ShapeDtypeStruct((NN, HID), F32),
                 jax.ShapeDtypeStruct((NN, H2), F32)),
            )(v, a0, a1, *nw, *puws[L + 1])
        else:
            v = _tc_call(
                _node_block_body, (NN // TN,),
                [_rows(TN, HID), _rows(TN, HID), _rows(TN, HID)] + nw_specs,
                _rows(TN, HID), jax.ShapeDtypeStruct((NN, HID), F32),
            )(v, a0, a1, *nw)

    dec = params["decoder"]
    return _tc_call(
        _dec_body, (NN // TN,),
        [_rows(TN, HID)] + _mlp_specs(HID, HID, OUT_DIM),
        _rows(TN, OUT_DIM), jax.ShapeDtypeStruct((NN, OUT_DIM), F32),
    )(v, *_mlp_args(dec))


# speedup vs baseline: 1.1346x; 1.1346x over previous
import functools

import jax
import jax.numpy as jnp
from jax import lax
from jax.experimental import pallas as pl
from jax.experimental.pallas import tpu as pltpu
from jax.experimental.pallas import tpu_sc as plsc

NN = 10000
NE = 320000
HID = 64
H2 = 128
OUT_DIM = 3

NC = 2
NS = 16
NW = NC * NS
EPW = NE // NW

GCH = 40
GGC = 2
GGROUP = GCH * GGC
GGN = EPW // GGROUP

SCH = 80
SGN = EPW // SCH

NPAD = 10240
NPS = NPAD // NS

F32 = jnp.float32


def _gather_body(uw_hbm, src_hbm, dst_hbm, g_hbm, si, di,
                 a0, b0, a1, b1, g0, g1, s0, s1):
    cid = lax.axis_index("c")
    sid = lax.axis_index("s")
    wid = cid * NS + sid
    pltpu.sync_copy(src_hbm.at[pl.ds(wid * EPW, EPW)], si)
    pltpu.sync_copy(dst_hbm.at[pl.ds(wid * EPW, EPW)], di)
    ebase = wid * EPW

    def issue(g, abuf, bbuf, sem):
        cps = []
        for k in range(GGC):
            off = (g * GGC + k) * GCH
            cps.append(pltpu.async_copy(
                uw_hbm.at[si.at[pl.ds(off, GCH)]],
                abuf.at[pl.ds(k * GCH, GCH)], sem))
            cps.append(pltpu.async_copy(
                uw_hbm.at[di.at[pl.ds(off, GCH)]],
                bbuf.at[pl.ds(k * GCH, GCH)], sem))
        return cps

    def finish(cps, g, abuf, bbuf, gbuf):
        for cp in cps:
            cp.wait()

        def addrow(r, c2):
            for c in range(HID // 16):
                gbuf[r, pl.ds(c * 16, 16)] = (
                    abuf[r, pl.ds(c * 16, 16)]
                    + bbuf[r, pl.ds(HID + c * 16, 16)])
            return c2

        lax.fori_loop(0, GGROUP, addrow, 0)
        pltpu.sync_copy(gbuf, g_hbm.at[pl.ds(ebase + g * GGROUP, GGROUP)])

    def pair(p, carry):
        cps0 = issue(2 * p, a0, b0, s0)
        cps1 = issue(2 * p + 1, a1, b1, s1)
        finish(cps0, 2 * p, a0, b0, g0)
        finish(cps1, 2 * p + 1, a1, b1, g1)
        return carry

    lax.fori_loop(0, GGN // 2, pair, 0)
    finish(issue(GGN - 1, a0, b0, s0), GGN - 1, a0, b0, g0)


def _sc_gather(uw, srcp, dstp):
    mesh = plsc.VectorSubcoreMesh(core_axis_name="c", subcore_axis_name="s")
    f = pl.kernel(
        _gather_body,
        out_type=jax.ShapeDtypeStruct((NE, HID), F32),
        mesh=mesh,
        scratch_types=[
            pltpu.VMEM((EPW,), jnp.int32),
            pltpu.VMEM((EPW,), jnp.int32),
            pltpu.VMEM((GGROUP, H2), F32),
            pltpu.VMEM((GGROUP, H2), F32),
            pltpu.VMEM((GGROUP, H2), F32),
            pltpu.VMEM((GGROUP, H2), F32),
            pltpu.VMEM((GGROUP, HID), F32),
            pltpu.VMEM((GGROUP, HID), F32),
            pltpu.SemaphoreType.DMA,
            pltpu.SemaphoreType.DMA,
        ],
    )
    return f(uw, srcp, dstp)


def _scatter_body(e_hbm, dst_hbm, z_hbm, a0_hbm, a1_hbm,
                  di, eb0, eb1, spmem, lsem0, lsem1, ssem):
    cid = lax.axis_index("c")
    sid = lax.axis_index("s")
    wid = cid * NS + sid
    pltpu.sync_copy(dst_hbm.at[wid], di)
    stripe = pl.ds(sid * NPS, NPS)
    pltpu.sync_copy(z_hbm.at[stripe], spmem.at[stripe])
    plsc.subcore_barrier()

    def erows(j):
        return e_hbm.at[pl.ds(wid * EPW + j * SCH, SCH), pl.ds(0, HID)]

    pltpu.sync_copy(erows(0), eb0)

    def step(p, carry):
        j = 2 * p
        ld1 = pltpu.async_copy(erows(j + 1), eb1, lsem1)
        pltpu.async_copy(eb0, spmem.at[di.at[j]], ssem, add=True).wait()
        ld1.wait()
        ld0 = pltpu.async_copy(erows(j + 2), eb0, lsem0)
        pltpu.async_copy(eb1, spmem.at[di.at[j + 1]], ssem, add=True).wait()
        ld0.wait()
        return carry

    lax.fori_loop(0, (SGN - 1) // 2, step, 0)
    pltpu.async_copy(eb0, spmem.at[di.at[SGN - 1]], ssem, add=True).wait()
    plsc.subcore_barrier()

    @pl.when(cid == 0)
    def _():
        pltpu.sync_copy(spmem.at[stripe], a0_hbm.at[stripe])

    @pl.when(cid == 1)
    def _():
        pltpu.sync_copy(spmem.at[stripe], a1_hbm.at[stripe])


def _sc_scatter(e, dst3, zeros):
    mesh = plsc.VectorSubcoreMesh(core_axis_name="c", subcore_axis_name="s")
    f = pl.kernel(
        _scatter_body,
        out_type=(jax.ShapeDtypeStruct((NPAD, HID), F32),
                  jax.ShapeDtypeStruct((NPAD, HID), F32)),
        mesh=mesh,
        scratch_types=[
            pltpu.VMEM((SGN, SCH), jnp.int32),
            pltpu.VMEM((SCH, HID), F32),
            pltpu.VMEM((SCH, HID), F32),
            pltpu.VMEM_SHARED((NPAD, HID), F32),
            pltpu.SemaphoreType.DMA,
            pltpu.SemaphoreType.DMA,
            pltpu.SemaphoreType.DMA,
        ],
        compiler_params=pltpu.CompilerParams(use_tc_tiling_on_sc=False),
    )
    return f(e, dst3, zeros)


def _dot(a, b):
    return jnp.dot(a, b, preferred_element_type=F32)


def _ln(h, gm, bt):
    m = jnp.mean(h, axis=-1, keepdims=True)
    d = h - m
    v = jnp.mean(d * d, axis=-1, keepdims=True)
    return d * lax.rsqrt(v + 1e-5) * gm + bt


def _pad128(x):
    return jnp.concatenate([x, jnp.zeros_like(x)], axis=-1)


def _edge_enc_body(x_ref, w0, b0, w1, b1, w2, b2, o_ref):
    h = jnp.maximum(_dot(x_ref[...], w0[...]) + b0[...], 0.0)
    h = jnp.maximum(_dot(h, w1[...]) + b1[...], 0.0)
    o_ref[...] = _pad128(_dot(h, w2[...]) + b2[...])


def _node_enc_body(x_ref, w0, b0, w1, b1, w2, b2, puw, pubw,
                   v_ref, uw_ref):
    h = jnp.maximum(_dot(x_ref[...], w0[...]) + b0[...], 0.0)
    h = jnp.maximum(_dot(h, w1[...]) + b1[...], 0.0)
    v = _dot(h, w2[...]) + b2[...]
    v_ref[...] = v
    uw_ref[...] = _dot(v, puw[...]) + pubw[...]


def _edge_block_body(g_ref, e_ref, w0e, w1, b1, w2, b2, gm, bt, o_ref):
    e = e_ref[...][:, :HID]
    h = jnp.maximum(g_ref[...] + _dot(e, w0e[...]), 0.0)
    h = jnp.maximum(_dot(h, w1[...]) + b1[...], 0.0)
    h = _dot(h, w2[...]) + b2[...]
    o_ref[...] = _pad128(e + _ln(h, gm[...], bt[...]))


def _node_block_body(v_ref, a0_ref, a1_ref, w0v, w0a, b0, w1, b1, w2, b2,
                     gm, bt, vo_ref):
    v = v_ref[...]
    a = a0_ref[...] + a1_ref[...]
    h = jnp.maximum(_dot(v, w0v[...]) + _dot(a, w0a[...]) + b0[...], 0.0)
    h = jnp.maximum(_dot(h, w1[...]) + b1[...], 0.0)
    h = _dot(h, w2[...]) + b2[...]
    vo_ref[...] = v + _ln(h, gm[...], bt[...])


def _node_block_proj_body(v_ref, a0_ref, a1_ref, w0v, w0a, b0, w1, b1,
                          w2, b2, gm, bt, puw, pubw,
                          vo_ref, uw_ref):
    v = v_ref[...]
    a = a0_ref[...] + a1_ref[...]
    h = jnp.maximum(_dot(v, w0v[...]) + _dot(a, w0a[...]) + b0[...], 0.0)
    h = jnp.maximum(_dot(h, w1[...]) + b1[...], 0.0)
    h = _dot(h, w2[...]) + b2[...]
    vn = v + _ln(h, gm[...], bt[...])
    vo_ref[...] = vn
    uw_ref[...] = _dot(vn, puw[...]) + pubw[...]


def _dec_body(v_ref, w0, b0, w1, b1, w2, b2, o_ref):
    h = jnp.maximum(_dot(v_ref[...], w0[...]) + b0[...], 0.0)
    h = jnp.maximum(_dot(h, w1[...]) + b1[...], 0.0)
    o_ref[...] = _dot(h, w2[...]) + b2[...]


def _full(shape):
    return pl.BlockSpec(shape, lambda i: (0,) * len(shape))


def _rows(t, d):
    return pl.BlockSpec((t, d), lambda i: (i, 0))


def _tc_call(body, grid, in_specs, out_specs, out_shape):
    return pl.pallas_call(
        body,
        grid=grid,
        in_specs=in_specs,
        out_specs=out_specs,
        out_shape=out_shape,
        compiler_params=pltpu.CompilerParams(
            dimension_semantics=("arbitrary",)),
    )


TE = 8000
TN = 2000


def _mlp_specs(in_dim, hid, out_dim):
    return [_full((in_dim, hid)), _full((1, hid)),
            _full((hid, hid)), _full((1, hid)),
            _full((hid, out_dim)), _full((1, out_dim))]


def _r1(x):
    return x.reshape(1, -1)


def _mlp_args(p):
    return (p["W0"], _r1(p["b0"]), p["W1"], _r1(p["b1"]), p["W2"], _r1(p["b2"]))


def kernel(node_feat, edge_feat, edge_index, params):
    srcp = edge_index[0]
    dstp = edge_index[1]
    dst3 = dstp.reshape(NW, SGN, SCH)

    ebs = params["edge_blocks"]
    nbs = params["node_blocks"]

    esplit = []
    puws = []
    for eb in ebs:
        W0 = eb["mlp"]["W0"]
        esplit.append(W0[2 * HID:])
        puw = jnp.concatenate([W0[:HID], W0[HID:2 * HID]], axis=1)
        pubw = jnp.concatenate(
            [eb["mlp"]["b0"], jnp.zeros((HID,), F32)]).reshape(1, H2)
        puws.append((puw, pubw))
    nsplit = []
    for nb in nbs:
        W0 = nb["mlp"]["W0"]
        nsplit.append((W0[:HID], W0[HID:]))

    ee = params["edge_enc"]
    e = _tc_call(
        _edge_enc_body, (NE // TE,),
        [_rows(TE, 16)] + _mlp_specs(16, HID, HID),
        _rows(TE, H2), jax.ShapeDtypeStruct((NE, H2), F32),
    )(edge_feat, *_mlp_args(ee))

    ne = params["node_enc"]
    v, uw = _tc_call(
        _node_enc_body, (NN // TN,),
        [_rows(TN, 128)] + _mlp_specs(128, HID, HID)
        + [_full((HID, H2)), _full((1, H2))],
        (_rows(TN, HID), _rows(TN, H2)),
        (jax.ShapeDtypeStruct((NN, HID), F32),
         jax.ShapeDtypeStruct((NN, H2), F32)),
    )(node_feat, *_mlp_args(ne), *puws[0])

    zeros = jnp.zeros((NPAD, HID), F32)

    for L in range(len(ebs)):
        eb, nb = ebs[L], nbs[L]
        g = _sc_gather(uw, srcp, dstp)
        e = _tc_call(
            _edge_block_body, (NE // TE,),
            [_rows(TE, HID), _rows(TE, H2)]
            + [_full((HID, HID)), _full((HID, HID)), _full((1, HID)),
               _full((HID, HID)), _full((1, HID)),
               _full((1, HID)), _full((1, HID))],
            _rows(TE, H2), jax.ShapeDtypeStruct((NE, H2), F32),
        )(g, e, esplit[L], eb["mlp"]["W1"], _r1(eb["mlp"]["b1"]),
          eb["mlp"]["W2"], _r1(eb["mlp"]["b2"]), _r1(eb["g"]), _r1(eb["b"]))

        a0, a1 = _sc_scatter(e, dst3, zeros)

        nw = (nsplit[L][0], nsplit[L][1], _r1(nb["mlp"]["b0"]),
              nb["mlp"]["W1"], _r1(nb["mlp"]["b1"]),
              nb["mlp"]["W2"], _r1(nb["mlp"]["b2"]),
              _r1(nb["g"]), _r1(nb["b"]))
        nw_specs = [_full((HID, HID)), _full((HID, HID)), _full((1, HID)),
                    _full((HID, HID)), _full((1, HID)),
                    _full((HID, HID)), _full((1, HID)),
                    _full((1, HID)), _full((1, HID))]
        if L + 1 < len(ebs):
            v, uw = _tc_call(
                _node_block_proj_body, (NN // TN,),
                [_rows(TN, HID), _rows(TN, HID), _rows(TN, HID)] + nw_specs
                + [_full((HID, H2)), _full((1, H2))],
                (_rows(TN, HID), _rows(TN, H2)),
                (jax.ShapeDtypeStruct((NN, HID), F32),
                 jax.ShapeDtypeStruct((NN, H2), F32)),
            )(v, a0, a1, *nw, *puws[L + 1])
        else:
            v = _tc_call(
                _node_block_body, (NN // TN,),
                [_rows(TN, HID), _rows(TN, HID), _rows(TN, HID)] + nw_specs,
                _rows(TN, HID), jax.ShapeDtypeStruct((NN, HID), F32),
            )(v, a0, a1, *nw)

    dec = params["decoder"]
    return _tc_call(
        _dec_body, (NN // TN,),
        [_rows(TN, HID)] + _mlp_specs(HID, HID, OUT_DIM),
        _rows(TN, OUT_DIM), jax.ShapeDtypeStruct((NN, OUT_DIM), F32),
    )(v, *_mlp_args(dec))
